# baseline (device time: 24521 ns/iter reference)
import jax
import jax.numpy as jnp
from jax import lax
from jax.experimental import pallas as pl
from jax.experimental.pallas import tpu as pltpu

CHUNKS = [64] * 14 + [48, 32, 16, 16, 16]
OFFS = [sum(CHUNKS[:i]) for i in range(len(CHUNKS))]
K = len(CHUNKS)


def kernel(x):
    _, m, n2 = x.shape
    n = n2 // 2
    half = m // 2
    assert sum(CHUNKS) == half

    def body(
        x_hbm,
        out_ref,
        xbuf,
        sybuf,
        rybuf,
        load_sems,
        ysend,
        yrecv,
        xsend,
        xrecv,
    ):
        my_x = lax.axis_index("x")
        my_y = lax.axis_index("y")
        peer_y = (my_x, 1 - my_y)
        peer_x = (1 - my_x, my_y)
        row0 = my_x * half

        def load_dma(k):
            return pltpu.make_async_copy(
                x_hbm.at[0, pl.ds(row0 + OFFS[k], CHUNKS[k]), :],
                xbuf.at[pl.ds(OFFS[k], CHUNKS[k]), :],
                load_sems.at[k],
            )

        def y_rdma(k):
            return pltpu.make_async_remote_copy(
                src_ref=sybuf.at[pl.ds(OFFS[k], CHUNKS[k]), :],
                dst_ref=rybuf.at[pl.ds(OFFS[k], CHUNKS[k]), :],
                send_sem=ysend.at[k],
                recv_sem=yrecv.at[k],
                device_id=peer_y,
                device_id_type=pl.DeviceIdType.MESH,
            )

        def x_rdma(k):
            return pltpu.make_async_remote_copy(
                src_ref=out_ref.at[pl.ds(row0 + OFFS[k], CHUNKS[k]), :],
                dst_ref=out_ref.at[pl.ds(row0 + OFFS[k], CHUNKS[k]), :],
                send_sem=xsend.at[k],
                recv_sem=xrecv.at[k],
                device_id=peer_x,
                device_id_type=pl.DeviceIdType.MESH,
            )

        def stage(k):
            load_dma(k).wait()

            @pl.when(my_y == 0)
            def _():
                sybuf[pl.ds(OFFS[k], CHUNKS[k]), :] = xbuf[
                    pl.ds(OFFS[k], CHUNKS[k]), n:
                ].astype(jnp.bfloat16)

            @pl.when(my_y == 1)
            def _():
                sybuf[pl.ds(OFFS[k], CHUNKS[k]), :] = xbuf[
                    pl.ds(OFFS[k], CHUNKS[k]), :n
                ].astype(jnp.bfloat16)

        for k in range(K):
            load_dma(k).start()
        stage(0)

        barrier = pltpu.get_barrier_semaphore()
        for p in (peer_y, peer_x):
            pl.semaphore_signal(
                barrier, inc=1, device_id=p, device_id_type=pl.DeviceIdType.MESH
            )
        pl.semaphore_wait(barrier, 2)

        y_rdma(0).start()
        for k in range(1, K):
            stage(k)
            y_rdma(k).start()

        def reduce_and_forward(k):
            rs = row0 + OFFS[k]
            y_rdma(k).wait_recv()

            @pl.when(my_y == 0)
            def _():
                out_ref[pl.ds(rs, CHUNKS[k]), :] = (
                    xbuf[pl.ds(OFFS[k], CHUNKS[k]), :n].astype(jnp.bfloat16)
                    + rybuf[pl.ds(OFFS[k], CHUNKS[k]), :]
                )

            @pl.when(my_y == 1)
            def _():
                out_ref[pl.ds(rs, CHUNKS[k]), :] = (
                    xbuf[pl.ds(OFFS[k], CHUNKS[k]), n:].astype(jnp.bfloat16)
                    + rybuf[pl.ds(OFFS[k], CHUNKS[k]), :]
                )

            x_rdma(k).start()

        for k in range(K):
            reduce_and_forward(k)

        for k in range(K):
            y_rdma(k).wait_send()
            x_rdma(k).wait_send()
            x_rdma(k).wait_recv()

    return pl.pallas_call(
        body,
        out_shape=jax.ShapeDtypeStruct((m, n), jnp.bfloat16),
        in_specs=[pl.BlockSpec(memory_space=pl.ANY)],
        out_specs=pl.BlockSpec(memory_space=pltpu.VMEM),
        scratch_shapes=[
            pltpu.VMEM((half, n2), jnp.float32),
            pltpu.VMEM((half, n), jnp.bfloat16),
            pltpu.VMEM((half, n), jnp.bfloat16),
            pltpu.SemaphoreType.DMA((K,)),
            pltpu.SemaphoreType.DMA((K,)),
            pltpu.SemaphoreType.DMA((K,)),
            pltpu.SemaphoreType.DMA((K,)),
            pltpu.SemaphoreType.DMA((K,)),
        ],
        compiler_params=pltpu.CompilerParams(collective_id=0),
    )(x)


# device time: 22606 ns/iter; 1.0847x vs baseline; 1.0847x over previous
import os

import jax
import jax.numpy as jnp
from jax import lax
from jax.experimental import pallas as pl
from jax.experimental.pallas import tpu as pltpu

_SKIP_X = os.environ.get("DIAG_SKIP_X") == "1"
_SKIP_Y = os.environ.get("DIAG_SKIP_Y") == "1"

CHUNKS = [64] * 14 + [48, 32, 16, 16, 16]
OFFS = [sum(CHUNKS[:i]) for i in range(len(CHUNKS))]
K = len(CHUNKS)


def kernel(x):
    _, m, n2 = x.shape
    n = n2 // 2
    half = m // 2
    assert sum(CHUNKS) == half

    def body(
        x_hbm,
        out_ref,
        xbuf,
        sybuf,
        rybuf,
        load_sems,
        ysend,
        yrecv,
        xsend,
        xrecv,
    ):
        my_x = lax.axis_index("x")
        my_y = lax.axis_index("y")
        peer_y = (my_x, 1 - my_y)
        peer_x = (1 - my_x, my_y)
        row0 = my_x * half

        def load_dma(k):
            return pltpu.make_async_copy(
                x_hbm.at[0, pl.ds(row0 + OFFS[k], CHUNKS[k]), :],
                xbuf.at[pl.ds(OFFS[k], CHUNKS[k]), :],
                load_sems.at[k],
            )

        def y_rdma(k):
            return pltpu.make_async_remote_copy(
                src_ref=sybuf.at[pl.ds(OFFS[k], CHUNKS[k]), :],
                dst_ref=rybuf.at[pl.ds(OFFS[k], CHUNKS[k]), :],
                send_sem=ysend.at[k],
                recv_sem=yrecv.at[k],
                device_id=peer_y,
                device_id_type=pl.DeviceIdType.MESH,
            )

        def x_rdma(k):
            return pltpu.make_async_remote_copy(
                src_ref=out_ref.at[pl.ds(row0 + OFFS[k], CHUNKS[k]), :],
                dst_ref=out_ref.at[pl.ds(row0 + OFFS[k], CHUNKS[k]), :],
                send_sem=xsend.at[k],
                recv_sem=xrecv.at[k],
                device_id=peer_x,
                device_id_type=pl.DeviceIdType.MESH,
            )

        def stage(k):
            load_dma(k).wait()

            @pl.when(my_y == 0)
            def _():
                sybuf[pl.ds(OFFS[k], CHUNKS[k]), :] = xbuf[
                    pl.ds(OFFS[k], CHUNKS[k]), n:
                ].astype(jnp.bfloat16)

            @pl.when(my_y == 1)
            def _():
                sybuf[pl.ds(OFFS[k], CHUNKS[k]), :] = xbuf[
                    pl.ds(OFFS[k], CHUNKS[k]), :n
                ].astype(jnp.bfloat16)

        for k in range(K):
            load_dma(k).start()
        stage(0)

        barrier = pltpu.get_barrier_semaphore()
        for p in (peer_y, peer_x):
            pl.semaphore_signal(
                barrier, inc=1, device_id=p, device_id_type=pl.DeviceIdType.MESH
            )
        pl.semaphore_wait(barrier, 2)

        if not _SKIP_Y:
            y_rdma(0).start()
            for k in range(1, K):
                stage(k)
                y_rdma(k).start()
        else:
            for k in range(1, K):
                stage(k)

        def reduce_and_forward(k):
            rs = row0 + OFFS[k]
            if not _SKIP_Y:
                y_rdma(k).wait_recv()

            @pl.when(my_y == 0)
            def _():
                out_ref[pl.ds(rs, CHUNKS[k]), :] = (
                    xbuf[pl.ds(OFFS[k], CHUNKS[k]), :n].astype(jnp.bfloat16)
                    + rybuf[pl.ds(OFFS[k], CHUNKS[k]), :]
                )

            @pl.when(my_y == 1)
            def _():
                out_ref[pl.ds(rs, CHUNKS[k]), :] = (
                    xbuf[pl.ds(OFFS[k], CHUNKS[k]), n:].astype(jnp.bfloat16)
                    + rybuf[pl.ds(OFFS[k], CHUNKS[k]), :]
                )

            if not _SKIP_X:
                x_rdma(k).start()

        for k in range(K):
            reduce_and_forward(k)

        for k in range(K):
            if not _SKIP_Y:
                y_rdma(k).wait_send()
            if not _SKIP_X:
                x_rdma(k).wait_send()
                x_rdma(k).wait_recv()

    return pl.pallas_call(
        body,
        out_shape=jax.ShapeDtypeStruct((m, n), jnp.bfloat16),
        in_specs=[pl.BlockSpec(memory_space=pl.ANY)],
        out_specs=pl.BlockSpec(memory_space=pltpu.VMEM),
        scratch_shapes=[
            pltpu.VMEM((half, n2), jnp.float32),
            pltpu.VMEM((half, n), jnp.bfloat16),
            pltpu.VMEM((half, n), jnp.bfloat16),
            pltpu.SemaphoreType.DMA((K,)),
            pltpu.SemaphoreType.DMA((K,)),
            pltpu.SemaphoreType.DMA((K,)),
            pltpu.SemaphoreType.DMA((K,)),
            pltpu.SemaphoreType.DMA((K,)),
        ],
        compiler_params=pltpu.CompilerParams(collective_id=0),
    )(x)
